# baseline (speedup 1.0000x reference)
import jax
import jax.numpy as jnp
from jax import lax
from jax.experimental import pallas as pl
from jax.experimental.pallas import tpu as pltpu
from jax.experimental.pallas import tpu_sc as plsc

BATCH = 16384
MAX_D = 20
DIM = 32

_B = BATCH * MAX_D
_NW = 32
_PER_W = _B // _NW
_BLK = 128
_NBLK = _PER_W // _BLK
_GRP = 8
_NGRP = _NBLK // _GRP


def _emb_body(idx_hbm, table_hbm, out_hbm, idx_v, rows_v, gsem, osem):
    cid = lax.axis_index("c")
    sid = lax.axis_index("s")
    wid = sid * 2 + cid
    base = wid * _PER_W

    pltpu.sync_copy(idx_hbm.at[pl.ds(base, _PER_W)], idx_v)

    def group(g, carry):
        copies = []
        for j in range(_GRP):
            copies.append(
                pltpu.async_copy(
                    table_hbm.at[idx_v.at[pl.ds((g * _GRP + j) * _BLK, _BLK)]],
                    rows_v.at[pl.ds(j * _BLK, _BLK)],
                    gsem,
                )
            )
        for c in copies:
            c.wait()
        pltpu.async_copy(
            rows_v, out_hbm.at[pl.ds(base + g * _GRP * _BLK, _GRP * _BLK)], osem
        ).wait()
        return carry

    lax.fori_loop(0, _NGRP, group, 0)


@jax.jit
def _emb_call(x_flat, table):
    mesh = plsc.VectorSubcoreMesh(core_axis_name="c", subcore_axis_name="s")
    f = pl.kernel(
        _emb_body,
        out_type=jax.ShapeDtypeStruct((_B, DIM), jnp.float32),
        mesh=mesh,
        scratch_types=[
            pltpu.VMEM((_PER_W,), jnp.int32),
            pltpu.VMEM((_GRP * _BLK, DIM), jnp.float32),
            pltpu.SemaphoreType.DMA,
            pltpu.SemaphoreType.DMA,
        ],
        compiler_params=pltpu.CompilerParams(use_tc_tiling_on_sc=False),
    )
    return f(x_flat, table)


def kernel(x, domain_emb_weight):
    x_flat = x.reshape(_B).astype(jnp.int32)
    out = _emb_call(x_flat, domain_emb_weight)
    return out.reshape(BATCH, MAX_D, DIM)
